# baseline (speedup 1.0000x reference)
import functools

import jax
import jax.numpy as jnp
from jax import lax
from jax.experimental import pallas as pl
from jax.experimental.pallas import tpu as pltpu
from jax.experimental.pallas import tpu_sc as plsc

_N = 10000
_E = 320000
_D = 128

_NC = 2
_NS = 16
_NW = _NC * _NS
_C = 25
_EPW = _E // _NW
_NCH = _EPW // _C
_RPT = _N // _NS
_NBUF = 5

@functools.cache
def _build_sc_seg_sum_deg():
    return functools.partial(
        pl.kernel,
        mesh=plsc.VectorSubcoreMesh(core_axis_name="c", subcore_axis_name="s"),
        compiler_params=pltpu.CompilerParams(use_tc_tiling_on_sc=False),
        out_type=(
            jax.ShapeDtypeStruct((_NW, _RPT, _D), jnp.float32),
            jax.ShapeDtypeStruct((_NC, _N), jnp.float32),
        ),
        scratch_types=[
            pltpu.VMEM((_NCH, _C), jnp.int32),
            pltpu.VMEM((_NCH, _C), jnp.int32),
        ] + [pltpu.VMEM((_C, _D), jnp.float32) for _ in range(_NBUF)] + [
            pltpu.VMEM((_C,), jnp.float32),
            pltpu.VMEM_SHARED((_N, _D), jnp.float32),
            pltpu.VMEM_SHARED((_N,), jnp.float32),
        ] + [pltpu.SemaphoreType.DMA for _ in range(_NBUF + 1)],
    )(_sc_seg_sum_deg_body)


def _sc_seg_sum_deg_body(x_hbm, ei_hbm, zn_hbm,
                         agg_out, deg_out,
                         src_v, dst_v, *rest):
    rows_b = rest[:_NBUF]
    ones_v = rest[_NBUF]
    acc_sh = rest[_NBUF + 1]
    deg_sh = rest[_NBUF + 2]
    sems = rest[_NBUF + 3:2 * _NBUF + 3]
    dsem = rest[2 * _NBUF + 3]
    r0b = rows_b[0]
    c = lax.axis_index("c")
    s = lax.axis_index("s")
    wid = c * _NS + s
    pltpu.sync_copy(ei_hbm.at[0, wid], src_v)
    pltpu.sync_copy(ei_hbm.at[1, wid], dst_v)

    @pl.when(s == 0)
    def _():
        pltpu.sync_copy(zn_hbm, deg_sh)

    for o in range(0, _C - 15, 16):
        ones_v[pl.ds(o, 16)] = jnp.ones((16,), jnp.float32)
    if _C % 16:
        ones_v[pl.ds(_C - 16, 16)] = jnp.ones((16,), jnp.float32)

    def _zrow(i, carry):
        for k in range(_D // 16):
            r0b[i, pl.ds(16 * k, 16)] = jnp.zeros((16,), jnp.float32)
        return carry

    lax.fori_loop(0, _C, _zrow, 0)
    r0 = s * _RPT
    for k in range(_RPT // _C):
        pltpu.sync_copy(r0b, acc_sh.at[pl.ds(r0 + k * _C, _C)])
    if _RPT % _C:
        pltpu.sync_copy(r0b.at[pl.ds(0, _RPT % _C)],
                        acc_sh.at[pl.ds(r0 + (_RPT // _C) * _C, _RPT % _C)])
    plsc.subcore_barrier()

    for b in range(_NBUF):
        pltpu.async_copy(x_hbm.at[src_v.at[b]], rows_b[b], sems[b])

    def _outer(g, carry):
        j0 = g * _NBUF
        for b in range(_NBUF):
            j = j0 + b
            pltpu.make_async_copy(x_hbm.at[src_v.at[j]], rows_b[b],
                                  sems[b]).wait()
            pltpu.sync_copy(rows_b[b], acc_sh.at[dst_v.at[j]], add=True)
            pltpu.async_copy(ones_v, deg_sh.at[dst_v.at[j]], dsem, add=True)
            nj = j + _NBUF

            @pl.when(nj < _NCH)
            def _():
                pltpu.async_copy(x_hbm.at[src_v.at[nj]], rows_b[b], sems[b])
        return carry

    lax.fori_loop(0, _NCH // _NBUF, _outer, 0)

    def _drain(j, carry):
        pltpu.make_async_copy(ones_v, deg_sh.at[dst_v.at[0]], dsem).wait()
        return carry

    lax.fori_loop(0, _NCH, _drain, 0)
    plsc.subcore_barrier()
    pltpu.sync_copy(acc_sh.at[pl.ds(r0, _RPT)], agg_out.at[wid])

    @pl.when(s == 0)
    def _():
        pltpu.sync_copy(deg_sh, deg_out.at[c])


@functools.cache
def _build_sc_seg_sum():
    return functools.partial(
        pl.kernel,
        mesh=plsc.VectorSubcoreMesh(core_axis_name="c", subcore_axis_name="s"),
        compiler_params=pltpu.CompilerParams(use_tc_tiling_on_sc=False),
        out_type=jax.ShapeDtypeStruct((_NW, _RPT, _D), jnp.float32),
        scratch_types=[
            pltpu.VMEM((_NCH, _C), jnp.int32),
            pltpu.VMEM((_NCH, _C), jnp.int32),
        ] + [pltpu.VMEM((_C, _D), jnp.float32) for _ in range(_NBUF)] + [
            pltpu.VMEM_SHARED((_N, _D), jnp.float32),
        ] + [pltpu.SemaphoreType.DMA for _ in range(_NBUF)],
    )(_sc_seg_sum_body)


def _sc_seg_sum_body(x_hbm, ei_hbm,
                     agg_out,
                     src_v, dst_v, *rest):
    rows_b = rest[:_NBUF]
    acc_sh = rest[_NBUF]
    sems = rest[_NBUF + 1:2 * _NBUF + 1]
    r0b = rows_b[0]
    c = lax.axis_index("c")
    s = lax.axis_index("s")
    wid = c * _NS + s
    pltpu.sync_copy(ei_hbm.at[0, wid], src_v)
    pltpu.sync_copy(ei_hbm.at[1, wid], dst_v)

    def _zrow(i, carry):
        for k in range(_D // 16):
            r0b[i, pl.ds(16 * k, 16)] = jnp.zeros((16,), jnp.float32)
        return carry

    lax.fori_loop(0, _C, _zrow, 0)
    r0 = s * _RPT
    for k in range(_RPT // _C):
        pltpu.sync_copy(r0b, acc_sh.at[pl.ds(r0 + k * _C, _C)])
    if _RPT % _C:
        pltpu.sync_copy(r0b.at[pl.ds(0, _RPT % _C)],
                        acc_sh.at[pl.ds(r0 + (_RPT // _C) * _C, _RPT % _C)])
    plsc.subcore_barrier()

    for b in range(_NBUF):
        pltpu.async_copy(x_hbm.at[src_v.at[b]], rows_b[b], sems[b])

    def _outer(g, carry):
        j0 = g * _NBUF
        for b in range(_NBUF):
            j = j0 + b
            pltpu.make_async_copy(x_hbm.at[src_v.at[j]], rows_b[b],
                                  sems[b]).wait()
            pltpu.sync_copy(rows_b[b], acc_sh.at[dst_v.at[j]], add=True)
            nj = j + _NBUF

            @pl.when(nj < _NCH)
            def _():
                pltpu.async_copy(x_hbm.at[src_v.at[nj]], rows_b[b], sems[b])
        return carry

    lax.fori_loop(0, _NCH // _NBUF, _outer, 0)
    plsc.subcore_barrier()
    pltpu.sync_copy(acc_sh.at[pl.ds(r0, _RPT)], agg_out.at[wid])


_B = 2000
_G = _N // _B


def _lin1_body(agg_ref, deg_ref, x_ref, wl_ref, wr_ref, b_ref,
               lin_ref, inv_ref, stats_ref):
    agg = agg_ref[0] + agg_ref[1]
    deg = deg_ref[0] + deg_ref[1]
    inv = 1.0 / jnp.maximum(deg, 1.0)
    mean = agg * inv
    lin = (lax.dot_general(mean, wl_ref[...], (((1,), (1,)), ((), ())),
                           preferred_element_type=jnp.float32)
           + lax.dot_general(x_ref[...], wr_ref[...], (((1,), (1,)), ((), ())),
                             preferred_element_type=jnp.float32)
           + b_ref[...])
    lin_ref[...] = lin
    inv_ref[...] = inv
    s0 = jnp.sum(lin, axis=0, keepdims=True)
    s1 = jnp.sum(lin * lin, axis=0, keepdims=True)
    part = jnp.concatenate([s0, s1, jnp.zeros((6, _D), jnp.float32)], axis=0)

    @pl.when(pl.program_id(0) == 0)
    def _():
        stats_ref[...] = part

    @pl.when(pl.program_id(0) != 0)
    def _():
        stats_ref[...] = stats_ref[...] + part


_lin1 = pl.pallas_call(
    _lin1_body,
    grid=(_G,),
    in_specs=[
        pl.BlockSpec((2, _B, _D), lambda i: (0, i, 0)),
        pl.BlockSpec((2, _B, 1), lambda i: (0, i, 0)),
        pl.BlockSpec((_B, _D), lambda i: (i, 0)),
        pl.BlockSpec((_D, _D), lambda i: (0, 0)),
        pl.BlockSpec((_D, _D), lambda i: (0, 0)),
        pl.BlockSpec((1, _D), lambda i: (0, 0)),
    ],
    out_specs=[
        pl.BlockSpec((_B, _D), lambda i: (i, 0)),
        pl.BlockSpec((_B, 1), lambda i: (i, 0)),
        pl.BlockSpec((8, _D), lambda i: (0, 0)),
    ],
    out_shape=[
        jax.ShapeDtypeStruct((_N, _D), jnp.float32),
        jax.ShapeDtypeStruct((_N, 1), jnp.float32),
        jax.ShapeDtypeStruct((8, _D), jnp.float32),
    ],
)


def _bn_body(lin_ref, stats_ref, g_ref, bta_ref, h_ref):
    st = stats_ref[...]
    mu = st[0:1, :] * (1.0 / _N)
    var = st[1:2, :] * (1.0 / _N) - mu * mu
    rstd = lax.rsqrt(var + 1e-5)
    h_ref[...] = jnp.maximum(
        (lin_ref[...] - mu) * (rstd * g_ref[...]) + bta_ref[...], 0.0)


_bn_relu = pl.pallas_call(
    _bn_body,
    grid=(_G,),
    in_specs=[
        pl.BlockSpec((_B, _D), lambda i: (i, 0)),
        pl.BlockSpec((8, _D), lambda i: (0, 0)),
        pl.BlockSpec((1, _D), lambda i: (0, 0)),
        pl.BlockSpec((1, _D), lambda i: (0, 0)),
    ],
    out_specs=pl.BlockSpec((_B, _D), lambda i: (i, 0)),
    out_shape=jax.ShapeDtypeStruct((_N, _D), jnp.float32),
)


def _fin_body(agg_ref, inv_ref, h_ref, wl_ref, wr_ref, b_ref, out_ref):
    mean = (agg_ref[0] + agg_ref[1]) * inv_ref[...]
    out_ref[...] = (lax.dot_general(mean, wl_ref[...], (((1,), (1,)), ((), ())),
                                    preferred_element_type=jnp.float32)
                    + lax.dot_general(h_ref[...], wr_ref[...],
                                      (((1,), (1,)), ((), ())),
                                      preferred_element_type=jnp.float32)
                    + b_ref[...])


_final = pl.pallas_call(
    _fin_body,
    grid=(_G,),
    in_specs=[
        pl.BlockSpec((2, _B, _D), lambda i: (0, i, 0)),
        pl.BlockSpec((_B, 1), lambda i: (i, 0)),
        pl.BlockSpec((_B, _D), lambda i: (i, 0)),
        pl.BlockSpec((_D, _D), lambda i: (0, 0)),
        pl.BlockSpec((_D, _D), lambda i: (0, 0)),
        pl.BlockSpec((1, _D), lambda i: (0, 0)),
    ],
    out_specs=pl.BlockSpec((_B, _D), lambda i: (i, 0)),
    out_shape=jax.ShapeDtypeStruct((_N, _D), jnp.float32),
)


def kernel(x, edge_index, W1_l, b1, W1_r, W2_l, b2, W2_r, bn_gamma, bn_beta):
    ei = edge_index.reshape(2, _NW, _NCH, _C)
    zn = jnp.zeros((_N,), jnp.float32)

    agg1_parts, deg_pair = _build_sc_seg_sum_deg()(x, ei, zn)
    agg1_pair = agg1_parts.reshape(_NC, _N, _D)
    lin, inv, stats = _lin1(agg1_pair, deg_pair.reshape(_NC, _N, 1), x,
                            W1_l, W1_r, b1.reshape(1, _D))
    h = _bn_relu(lin, stats, bn_gamma.reshape(1, _D), bn_beta.reshape(1, _D))
    agg2_pair = _build_sc_seg_sum()(h, ei).reshape(_NC, _N, _D)
    out = _final(agg2_pair, inv, h, W2_l, W2_r, b2.reshape(1, _D))
    return out


# speedup vs baseline: 1.2184x; 1.2184x over previous
import functools

import jax
import jax.numpy as jnp
from jax import lax
from jax.experimental import pallas as pl
from jax.experimental.pallas import tpu as pltpu
from jax.experimental.pallas import tpu_sc as plsc

_N = 10000
_E = 320000
_D = 128

_NC = 2
_NS = 16
_NW = _NC * _NS
_C = 40
_EPW = _E // _NW
_NCH = _EPW // _C
_RPT = _N // _NS
_NBUF = 5

@functools.cache
def _build_sc_seg_sum_deg():
    return functools.partial(
        pl.kernel,
        mesh=plsc.VectorSubcoreMesh(core_axis_name="c", subcore_axis_name="s"),
        compiler_params=pltpu.CompilerParams(use_tc_tiling_on_sc=False),
        out_type=(
            jax.ShapeDtypeStruct((_NW, _RPT, _D), jnp.float32),
            jax.ShapeDtypeStruct((_NC, _N), jnp.float32),
        ),
        scratch_types=[
            pltpu.VMEM((_NCH, _C), jnp.int32),
            pltpu.VMEM((_NCH, _C), jnp.int32),
        ] + [pltpu.VMEM((_C, _D), jnp.float32) for _ in range(_NBUF)] + [
            pltpu.VMEM((_C,), jnp.float32),
            pltpu.VMEM_SHARED((_N, _D), jnp.float32),
            pltpu.VMEM_SHARED((_N,), jnp.float32),
        ] + [pltpu.SemaphoreType.DMA for _ in range(2 * _NBUF + 1)],
    )(_sc_seg_sum_deg_body)


def _sc_seg_sum_deg_body(x_hbm, ei_hbm, zn_hbm,
                         agg_out, deg_out,
                         src_v, dst_v, *rest):
    rows_b = rest[:_NBUF]
    ones_v = rest[_NBUF]
    acc_sh = rest[_NBUF + 1]
    deg_sh = rest[_NBUF + 2]
    sems = rest[_NBUF + 3:2 * _NBUF + 3]
    ssems = rest[2 * _NBUF + 3:3 * _NBUF + 3]
    dsem = rest[3 * _NBUF + 3]
    r0b = rows_b[0]
    c = lax.axis_index("c")
    s = lax.axis_index("s")
    wid = c * _NS + s
    pltpu.sync_copy(ei_hbm.at[0, wid], src_v)
    pltpu.sync_copy(ei_hbm.at[1, wid], dst_v)

    @pl.when(s == 0)
    def _():
        pltpu.sync_copy(zn_hbm, deg_sh)

    for o in range(0, _C - 15, 16):
        ones_v[pl.ds(o, 16)] = jnp.ones((16,), jnp.float32)
    if _C % 16:
        ones_v[pl.ds(_C - 16, 16)] = jnp.ones((16,), jnp.float32)

    def _zrow(i, carry):
        for k in range(_D // 16):
            r0b[i, pl.ds(16 * k, 16)] = jnp.zeros((16,), jnp.float32)
        return carry

    lax.fori_loop(0, _C, _zrow, 0)
    r0 = s * _RPT
    for k in range(_RPT // _C):
        pltpu.sync_copy(r0b, acc_sh.at[pl.ds(r0 + k * _C, _C)])
    if _RPT % _C:
        pltpu.sync_copy(r0b.at[pl.ds(0, _RPT % _C)],
                        acc_sh.at[pl.ds(r0 + (_RPT // _C) * _C, _RPT % _C)])
    plsc.subcore_barrier()

    for b in range(_NBUF):
        pltpu.async_copy(x_hbm.at[src_v.at[b]], rows_b[b], sems[b])

    def _outer(g, carry):
        j0 = g * _NBUF
        for b in range(_NBUF):
            j = j0 + b
            pltpu.make_async_copy(x_hbm.at[src_v.at[j]], rows_b[b],
                                  sems[b]).wait()
            pltpu.async_copy(rows_b[b], acc_sh.at[dst_v.at[j]], ssems[b],
                             add=True)
            pltpu.async_copy(ones_v, deg_sh.at[dst_v.at[j]], dsem, add=True)
            pb = (b - 1) % _NBUF
            pj = j - 1
            nj = pj + _NBUF

            @pl.when((pj >= 0) & (nj < _NCH))
            def _():
                pltpu.make_async_copy(rows_b[pb], acc_sh.at[dst_v.at[0]],
                                      ssems[pb]).wait()
                pltpu.async_copy(x_hbm.at[src_v.at[nj]], rows_b[pb], sems[pb])
        return carry

    lax.fori_loop(0, _NCH // _NBUF, _outer, 0)

    for b in range(_NBUF):
        pltpu.make_async_copy(rows_b[b], acc_sh.at[dst_v.at[0]],
                              ssems[b]).wait()

    def _drain(j, carry):
        pltpu.make_async_copy(ones_v, deg_sh.at[dst_v.at[0]], dsem).wait()
        return carry

    lax.fori_loop(0, _NCH, _drain, 0)
    plsc.subcore_barrier()
    pltpu.sync_copy(acc_sh.at[pl.ds(r0, _RPT)], agg_out.at[wid])

    @pl.when(s == 0)
    def _():
        pltpu.sync_copy(deg_sh, deg_out.at[c])


@functools.cache
def _build_sc_seg_sum():
    return functools.partial(
        pl.kernel,
        mesh=plsc.VectorSubcoreMesh(core_axis_name="c", subcore_axis_name="s"),
        compiler_params=pltpu.CompilerParams(use_tc_tiling_on_sc=False),
        out_type=jax.ShapeDtypeStruct((_NW, _RPT, _D), jnp.float32),
        scratch_types=[
            pltpu.VMEM((_NCH, _C), jnp.int32),
            pltpu.VMEM((_NCH, _C), jnp.int32),
        ] + [pltpu.VMEM((_C, _D), jnp.float32) for _ in range(_NBUF)] + [
            pltpu.VMEM_SHARED((_N, _D), jnp.float32),
        ] + [pltpu.SemaphoreType.DMA for _ in range(2 * _NBUF)],
    )(_sc_seg_sum_body)


def _sc_seg_sum_body(x_hbm, ei_hbm,
                     agg_out,
                     src_v, dst_v, *rest):
    rows_b = rest[:_NBUF]
    acc_sh = rest[_NBUF]
    sems = rest[_NBUF + 1:2 * _NBUF + 1]
    ssems = rest[2 * _NBUF + 1:3 * _NBUF + 1]
    r0b = rows_b[0]
    c = lax.axis_index("c")
    s = lax.axis_index("s")
    wid = c * _NS + s
    pltpu.sync_copy(ei_hbm.at[0, wid], src_v)
    pltpu.sync_copy(ei_hbm.at[1, wid], dst_v)

    def _zrow(i, carry):
        for k in range(_D // 16):
            r0b[i, pl.ds(16 * k, 16)] = jnp.zeros((16,), jnp.float32)
        return carry

    lax.fori_loop(0, _C, _zrow, 0)
    r0 = s * _RPT
    for k in range(_RPT // _C):
        pltpu.sync_copy(r0b, acc_sh.at[pl.ds(r0 + k * _C, _C)])
    if _RPT % _C:
        pltpu.sync_copy(r0b.at[pl.ds(0, _RPT % _C)],
                        acc_sh.at[pl.ds(r0 + (_RPT // _C) * _C, _RPT % _C)])
    plsc.subcore_barrier()

    for b in range(_NBUF):
        pltpu.async_copy(x_hbm.at[src_v.at[b]], rows_b[b], sems[b])

    def _outer(g, carry):
        j0 = g * _NBUF
        for b in range(_NBUF):
            j = j0 + b
            pltpu.make_async_copy(x_hbm.at[src_v.at[j]], rows_b[b],
                                  sems[b]).wait()
            pltpu.async_copy(rows_b[b], acc_sh.at[dst_v.at[j]], ssems[b],
                             add=True)
            pb = (b - 1) % _NBUF
            pj = j - 1
            nj = pj + _NBUF

            @pl.when((pj >= 0) & (nj < _NCH))
            def _():
                pltpu.make_async_copy(rows_b[pb], acc_sh.at[dst_v.at[0]],
                                      ssems[pb]).wait()
                pltpu.async_copy(x_hbm.at[src_v.at[nj]], rows_b[pb], sems[pb])
        return carry

    lax.fori_loop(0, _NCH // _NBUF, _outer, 0)
    for b in range(_NBUF):
        pltpu.make_async_copy(rows_b[b], acc_sh.at[dst_v.at[0]],
                              ssems[b]).wait()
    plsc.subcore_barrier()
    pltpu.sync_copy(acc_sh.at[pl.ds(r0, _RPT)], agg_out.at[wid])


_B = 2000
_G = _N // _B


def _lin1_body(agg_ref, deg_ref, x_ref, wl_ref, wr_ref, b_ref,
               lin_ref, inv_ref, stats_ref):
    agg = agg_ref[0] + agg_ref[1]
    deg = deg_ref[0] + deg_ref[1]
    inv = 1.0 / jnp.maximum(deg, 1.0)
    mean = agg * inv
    lin = (lax.dot_general(mean, wl_ref[...], (((1,), (1,)), ((), ())),
                           preferred_element_type=jnp.float32)
           + lax.dot_general(x_ref[...], wr_ref[...], (((1,), (1,)), ((), ())),
                             preferred_element_type=jnp.float32)
           + b_ref[...])
    lin_ref[...] = lin
    inv_ref[...] = inv
    s0 = jnp.sum(lin, axis=0, keepdims=True)
    s1 = jnp.sum(lin * lin, axis=0, keepdims=True)
    part = jnp.concatenate([s0, s1, jnp.zeros((6, _D), jnp.float32)], axis=0)

    @pl.when(pl.program_id(0) == 0)
    def _():
        stats_ref[...] = part

    @pl.when(pl.program_id(0) != 0)
    def _():
        stats_ref[...] = stats_ref[...] + part


_lin1 = pl.pallas_call(
    _lin1_body,
    grid=(_G,),
    in_specs=[
        pl.BlockSpec((2, _B, _D), lambda i: (0, i, 0)),
        pl.BlockSpec((2, _B, 1), lambda i: (0, i, 0)),
        pl.BlockSpec((_B, _D), lambda i: (i, 0)),
        pl.BlockSpec((_D, _D), lambda i: (0, 0)),
        pl.BlockSpec((_D, _D), lambda i: (0, 0)),
        pl.BlockSpec((1, _D), lambda i: (0, 0)),
    ],
    out_specs=[
        pl.BlockSpec((_B, _D), lambda i: (i, 0)),
        pl.BlockSpec((_B, 1), lambda i: (i, 0)),
        pl.BlockSpec((8, _D), lambda i: (0, 0)),
    ],
    out_shape=[
        jax.ShapeDtypeStruct((_N, _D), jnp.float32),
        jax.ShapeDtypeStruct((_N, 1), jnp.float32),
        jax.ShapeDtypeStruct((8, _D), jnp.float32),
    ],
)


def _bn_body(lin_ref, stats_ref, g_ref, bta_ref, h_ref):
    st = stats_ref[...]
    mu = st[0:1, :] * (1.0 / _N)
    var = st[1:2, :] * (1.0 / _N) - mu * mu
    rstd = lax.rsqrt(var + 1e-5)
    h_ref[...] = jnp.maximum(
        (lin_ref[...] - mu) * (rstd * g_ref[...]) + bta_ref[...], 0.0)


_bn_relu = pl.pallas_call(
    _bn_body,
    grid=(_G,),
    in_specs=[
        pl.BlockSpec((_B, _D), lambda i: (i, 0)),
        pl.BlockSpec((8, _D), lambda i: (0, 0)),
        pl.BlockSpec((1, _D), lambda i: (0, 0)),
        pl.BlockSpec((1, _D), lambda i: (0, 0)),
    ],
    out_specs=pl.BlockSpec((_B, _D), lambda i: (i, 0)),
    out_shape=jax.ShapeDtypeStruct((_N, _D), jnp.float32),
)


def _fin_body(agg_ref, inv_ref, h_ref, wl_ref, wr_ref, b_ref, out_ref):
    mean = (agg_ref[0] + agg_ref[1]) * inv_ref[...]
    out_ref[...] = (lax.dot_general(mean, wl_ref[...], (((1,), (1,)), ((), ())),
                                    preferred_element_type=jnp.float32)
                    + lax.dot_general(h_ref[...], wr_ref[...],
                                      (((1,), (1,)), ((), ())),
                                      preferred_element_type=jnp.float32)
                    + b_ref[...])


_final = pl.pallas_call(
    _fin_body,
    grid=(_G,),
    in_specs=[
        pl.BlockSpec((2, _B, _D), lambda i: (0, i, 0)),
        pl.BlockSpec((_B, 1), lambda i: (i, 0)),
        pl.BlockSpec((_B, _D), lambda i: (i, 0)),
        pl.BlockSpec((_D, _D), lambda i: (0, 0)),
        pl.BlockSpec((_D, _D), lambda i: (0, 0)),
        pl.BlockSpec((1, _D), lambda i: (0, 0)),
    ],
    out_specs=pl.BlockSpec((_B, _D), lambda i: (i, 0)),
    out_shape=jax.ShapeDtypeStruct((_N, _D), jnp.float32),
)


def kernel(x, edge_index, W1_l, b1, W1_r, W2_l, b2, W2_r, bn_gamma, bn_beta):
    ei = edge_index.reshape(2, _NW, _NCH, _C)
    zn = jnp.zeros((_N,), jnp.float32)

    agg1_parts, deg_pair = _build_sc_seg_sum_deg()(x, ei, zn)
    agg1_pair = agg1_parts.reshape(_NC, _N, _D)
    lin, inv, stats = _lin1(agg1_pair, deg_pair.reshape(_NC, _N, 1), x,
                            W1_l, W1_r, b1.reshape(1, _D))
    h = _bn_relu(lin, stats, bn_gamma.reshape(1, _D), bn_beta.reshape(1, _D))
    agg2_pair = _build_sc_seg_sum()(h, ei).reshape(_NC, _N, _D)
    out = _final(agg2_pair, inv, h, W2_l, W2_r, b2.reshape(1, _D))
    return out


# speedup vs baseline: 1.2477x; 1.0240x over previous
import functools

import jax
import jax.numpy as jnp
from jax import lax
from jax.experimental import pallas as pl
from jax.experimental.pallas import tpu as pltpu
from jax.experimental.pallas import tpu_sc as plsc

_N = 10000
_E = 320000
_D = 128

_NC = 2
_NS = 16
_NW = _NC * _NS
_C = 40
_EPW = _E // _NW
_NCH = _EPW // _C
_RPT = _N // _NS
_NBUF = 5

@functools.cache
def _build_sc_seg_sum_deg():
    return functools.partial(
        pl.kernel,
        mesh=plsc.VectorSubcoreMesh(core_axis_name="c", subcore_axis_name="s"),
        compiler_params=pltpu.CompilerParams(use_tc_tiling_on_sc=False),
        out_type=(
            jax.ShapeDtypeStruct((_NW, _RPT, _D), jnp.float32),
            jax.ShapeDtypeStruct((_NC, _N), jnp.float32),
        ),
        scratch_types=[
            pltpu.VMEM((_NCH, _C), jnp.int32),
            pltpu.VMEM((_NCH, _C), jnp.int32),
        ] + [pltpu.VMEM((_C, _D), jnp.float32) for _ in range(_NBUF)] + [
            pltpu.VMEM((_C,), jnp.float32),
            pltpu.VMEM_SHARED((_N, _D), jnp.float32),
            pltpu.VMEM_SHARED((_N,), jnp.float32),
        ] + [pltpu.SemaphoreType.DMA for _ in range(_NBUF + 1)],
    )(_sc_seg_sum_deg_body)


def _sc_seg_sum_deg_body(x_hbm, ei_hbm, zn_hbm,
                         agg_out, deg_out,
                         src_v, dst_v, *rest):
    rows_b = rest[:_NBUF]
    ones_v = rest[_NBUF]
    acc_sh = rest[_NBUF + 1]
    deg_sh = rest[_NBUF + 2]
    sems = rest[_NBUF + 3:2 * _NBUF + 3]
    dsem = rest[2 * _NBUF + 3]
    r0b = rows_b[0]
    c = lax.axis_index("c")
    s = lax.axis_index("s")
    wid = c * _NS + s
    pltpu.sync_copy(ei_hbm.at[0, wid], src_v)
    pltpu.sync_copy(ei_hbm.at[1, wid], dst_v)

    @pl.when(s == 0)
    def _():
        pltpu.sync_copy(zn_hbm, deg_sh)

    for o in range(0, _C - 15, 16):
        ones_v[pl.ds(o, 16)] = jnp.ones((16,), jnp.float32)
    if _C % 16:
        ones_v[pl.ds(_C - 16, 16)] = jnp.ones((16,), jnp.float32)

    def _zrow(i, carry):
        for k in range(_D // 16):
            r0b[i, pl.ds(16 * k, 16)] = jnp.zeros((16,), jnp.float32)
        return carry

    lax.fori_loop(0, _C, _zrow, 0)
    r0 = s * _RPT
    for k in range(_RPT // _C):
        pltpu.sync_copy(r0b, acc_sh.at[pl.ds(r0 + k * _C, _C)])
    if _RPT % _C:
        pltpu.sync_copy(r0b.at[pl.ds(0, _RPT % _C)],
                        acc_sh.at[pl.ds(r0 + (_RPT // _C) * _C, _RPT % _C)])
    plsc.subcore_barrier()

    for b in range(_NBUF):
        pltpu.async_copy(x_hbm.at[src_v.at[b]], rows_b[b], sems[b])

    def _outer(g, carry):
        j0 = g * _NBUF
        for b in range(_NBUF):
            j = j0 + b
            pltpu.make_async_copy(x_hbm.at[src_v.at[j]], rows_b[b],
                                  sems[b]).wait()
            pltpu.sync_copy(rows_b[b], acc_sh.at[dst_v.at[j]], add=True)
            pltpu.async_copy(ones_v, deg_sh.at[dst_v.at[j]], dsem, add=True)
            nj = j + _NBUF

            @pl.when(nj < _NCH)
            def _():
                pltpu.async_copy(x_hbm.at[src_v.at[nj]], rows_b[b], sems[b])
        return carry

    lax.fori_loop(0, _NCH // _NBUF, _outer, 0)

    def _drain(j, carry):
        pltpu.make_async_copy(ones_v, deg_sh.at[dst_v.at[0]], dsem).wait()
        return carry

    lax.fori_loop(0, _NCH, _drain, 0)
    plsc.subcore_barrier()
    pltpu.sync_copy(acc_sh.at[pl.ds(r0, _RPT)], agg_out.at[wid])

    @pl.when(s == 0)
    def _():
        pltpu.sync_copy(deg_sh, deg_out.at[c])


@functools.cache
def _build_sc_seg_sum():
    return functools.partial(
        pl.kernel,
        mesh=plsc.VectorSubcoreMesh(core_axis_name="c", subcore_axis_name="s"),
        compiler_params=pltpu.CompilerParams(use_tc_tiling_on_sc=False),
        out_type=jax.ShapeDtypeStruct((_NW, _RPT, _D), jnp.float32),
        scratch_types=[
            pltpu.VMEM((_NCH, _C), jnp.int32),
            pltpu.VMEM((_NCH, _C), jnp.int32),
        ] + [pltpu.VMEM((_C, _D), jnp.float32) for _ in range(_NBUF)] + [
            pltpu.VMEM_SHARED((_N, _D), jnp.float32),
        ] + [pltpu.SemaphoreType.DMA for _ in range(_NBUF)],
    )(_sc_seg_sum_body)


def _sc_seg_sum_body(x_hbm, ei_hbm,
                     agg_out,
                     src_v, dst_v, *rest):
    rows_b = rest[:_NBUF]
    acc_sh = rest[_NBUF]
    sems = rest[_NBUF + 1:2 * _NBUF + 1]
    r0b = rows_b[0]
    c = lax.axis_index("c")
    s = lax.axis_index("s")
    wid = c * _NS + s
    pltpu.sync_copy(ei_hbm.at[0, wid], src_v)
    pltpu.sync_copy(ei_hbm.at[1, wid], dst_v)

    def _zrow(i, carry):
        for k in range(_D // 16):
            r0b[i, pl.ds(16 * k, 16)] = jnp.zeros((16,), jnp.float32)
        return carry

    lax.fori_loop(0, _C, _zrow, 0)
    r0 = s * _RPT
    for k in range(_RPT // _C):
        pltpu.sync_copy(r0b, acc_sh.at[pl.ds(r0 + k * _C, _C)])
    if _RPT % _C:
        pltpu.sync_copy(r0b.at[pl.ds(0, _RPT % _C)],
                        acc_sh.at[pl.ds(r0 + (_RPT // _C) * _C, _RPT % _C)])
    plsc.subcore_barrier()

    for b in range(_NBUF):
        pltpu.async_copy(x_hbm.at[src_v.at[b]], rows_b[b], sems[b])

    def _outer(g, carry):
        j0 = g * _NBUF
        for b in range(_NBUF):
            j = j0 + b
            pltpu.make_async_copy(x_hbm.at[src_v.at[j]], rows_b[b],
                                  sems[b]).wait()
            pltpu.sync_copy(rows_b[b], acc_sh.at[dst_v.at[j]], add=True)
            nj = j + _NBUF

            @pl.when(nj < _NCH)
            def _():
                pltpu.async_copy(x_hbm.at[src_v.at[nj]], rows_b[b], sems[b])
        return carry

    lax.fori_loop(0, _NCH // _NBUF, _outer, 0)
    plsc.subcore_barrier()
    pltpu.sync_copy(acc_sh.at[pl.ds(r0, _RPT)], agg_out.at[wid])


_B = 2000
_G = _N // _B


def _lin1_body(agg_ref, deg_ref, x_ref, wl_ref, wr_ref, b_ref,
               lin_ref, inv_ref, stats_ref):
    agg = agg_ref[0] + agg_ref[1]
    deg = deg_ref[0] + deg_ref[1]
    inv = 1.0 / jnp.maximum(deg, 1.0)
    mean = agg * inv
    lin = (lax.dot_general(mean, wl_ref[...], (((1,), (1,)), ((), ())),
                           preferred_element_type=jnp.float32)
           + lax.dot_general(x_ref[...], wr_ref[...], (((1,), (1,)), ((), ())),
                             preferred_element_type=jnp.float32)
           + b_ref[...])
    lin_ref[...] = lin
    inv_ref[...] = inv
    s0 = jnp.sum(lin, axis=0, keepdims=True)
    s1 = jnp.sum(lin * lin, axis=0, keepdims=True)
    part = jnp.concatenate([s0, s1, jnp.zeros((6, _D), jnp.float32)], axis=0)

    @pl.when(pl.program_id(0) == 0)
    def _():
        stats_ref[...] = part

    @pl.when(pl.program_id(0) != 0)
    def _():
        stats_ref[...] = stats_ref[...] + part


_lin1 = pl.pallas_call(
    _lin1_body,
    grid=(_G,),
    in_specs=[
        pl.BlockSpec((2, _B, _D), lambda i: (0, i, 0)),
        pl.BlockSpec((2, _B, 1), lambda i: (0, i, 0)),
        pl.BlockSpec((_B, _D), lambda i: (i, 0)),
        pl.BlockSpec((_D, _D), lambda i: (0, 0)),
        pl.BlockSpec((_D, _D), lambda i: (0, 0)),
        pl.BlockSpec((1, _D), lambda i: (0, 0)),
    ],
    out_specs=[
        pl.BlockSpec((_B, _D), lambda i: (i, 0)),
        pl.BlockSpec((_B, 1), lambda i: (i, 0)),
        pl.BlockSpec((8, _D), lambda i: (0, 0)),
    ],
    out_shape=[
        jax.ShapeDtypeStruct((_N, _D), jnp.float32),
        jax.ShapeDtypeStruct((_N, 1), jnp.float32),
        jax.ShapeDtypeStruct((8, _D), jnp.float32),
    ],
)


def _bn_body(lin_ref, stats_ref, g_ref, bta_ref, h_ref):
    st = stats_ref[...]
    mu = st[0:1, :] * (1.0 / _N)
    var = st[1:2, :] * (1.0 / _N) - mu * mu
    rstd = lax.rsqrt(var + 1e-5)
    h_ref[...] = jnp.maximum(
        (lin_ref[...] - mu) * (rstd * g_ref[...]) + bta_ref[...], 0.0)


_bn_relu = pl.pallas_call(
    _bn_body,
    grid=(_G,),
    in_specs=[
        pl.BlockSpec((_B, _D), lambda i: (i, 0)),
        pl.BlockSpec((8, _D), lambda i: (0, 0)),
        pl.BlockSpec((1, _D), lambda i: (0, 0)),
        pl.BlockSpec((1, _D), lambda i: (0, 0)),
    ],
    out_specs=pl.BlockSpec((_B, _D), lambda i: (i, 0)),
    out_shape=jax.ShapeDtypeStruct((_N, _D), jnp.float32),
)


def _fin_body(agg_ref, inv_ref, h_ref, wl_ref, wr_ref, b_ref, out_ref):
    mean = (agg_ref[0] + agg_ref[1]) * inv_ref[...]
    out_ref[...] = (lax.dot_general(mean, wl_ref[...], (((1,), (1,)), ((), ())),
                                    preferred_element_type=jnp.float32)
                    + lax.dot_general(h_ref[...], wr_ref[...],
                                      (((1,), (1,)), ((), ())),
                                      preferred_element_type=jnp.float32)
                    + b_ref[...])


_final = pl.pallas_call(
    _fin_body,
    grid=(_G,),
    in_specs=[
        pl.BlockSpec((2, _B, _D), lambda i: (0, i, 0)),
        pl.BlockSpec((_B, 1), lambda i: (i, 0)),
        pl.BlockSpec((_B, _D), lambda i: (i, 0)),
        pl.BlockSpec((_D, _D), lambda i: (0, 0)),
        pl.BlockSpec((_D, _D), lambda i: (0, 0)),
        pl.BlockSpec((1, _D), lambda i: (0, 0)),
    ],
    out_specs=pl.BlockSpec((_B, _D), lambda i: (i, 0)),
    out_shape=jax.ShapeDtypeStruct((_N, _D), jnp.float32),
)


def kernel(x, edge_index, W1_l, b1, W1_r, W2_l, b2, W2_r, bn_gamma, bn_beta):
    ei = edge_index.reshape(2, _NW, _NCH, _C)
    zn = jnp.zeros((_N,), jnp.float32)

    agg1_parts, deg_pair = _build_sc_seg_sum_deg()(x, ei, zn)
    agg1_pair = agg1_parts.reshape(_NC, _N, _D)
    lin, inv, stats = _lin1(agg1_pair, deg_pair.reshape(_NC, _N, 1), x,
                            W1_l, W1_r, b1.reshape(1, _D))
    h = _bn_relu(lin, stats, bn_gamma.reshape(1, _D), bn_beta.reshape(1, _D))
    agg2_pair = _build_sc_seg_sum()(h, ei).reshape(_NC, _N, _D)
    out = _final(agg2_pair, inv, h, W2_l, W2_r, b2.reshape(1, _D))
    return out
